# baseline (speedup 1.0000x reference)
import jax
import jax.numpy as jnp
from jax import lax
from jax.experimental import pallas as pl
from jax.experimental.pallas import tpu as pltpu

_B = 16384
_D_IN = 768
_D_H = 2048
_D_E = 256
_K = 1024
_L = 3
_COMMIT = 0.25

_BM = 1024
_NB = _B // _BM


_NCH = 2
_CH = _BM // _NCH


def _fused_body(x_ref, w1_ref, b1_ref, w2_ref, b2_ref,
                dw1_ref, db1_ref, dw2_ref, db2_ref,
                cbt_ref, cbt2_ref, g_ref, out_ref, c2_ref):
    i = pl.program_id(0)

    @pl.when(i == 0)
    def _precompute_c2():
        for l in range(_L):
            cbt32 = cbt_ref[l].astype(jnp.float32)
            c2_ref[l] = jnp.sum(cbt32 * cbt32, axis=0, keepdims=True)
    part = jnp.zeros((), jnp.float32)
    for c in range(_NCH):
        x = x_ref[c * _CH:(c + 1) * _CH]
        xb = x.astype(jnp.bfloat16)
        h = jnp.maximum(
            jnp.dot(xb, w1_ref[...], preferred_element_type=jnp.float32)
            + b1_ref[...], 0.0)
        res0 = (jnp.dot(h.astype(jnp.bfloat16), w2_ref[...],
                        preferred_element_type=jnp.float32)
                + b2_ref[...])
        res = res0
        rq = jnp.zeros((_CH, 1), jnp.float32)
        for l in range(_L):
            s = (jnp.dot(res.astype(jnp.bfloat16), cbt2_ref[l],
                         preferred_element_type=jnp.float32)
                 - c2_ref[l]
                 + g_ref[l, c * _CH:(c + 1) * _CH]).astype(jnp.bfloat16)
            m = jnp.max(s, axis=1, keepdims=True)
            oh = (s >= m).astype(jnp.bfloat16)
            emb = lax.dot_general(oh, cbt_ref[l],
                                  (((1,), (1,)), ((), ())),
                                  preferred_element_type=jnp.float32)
            res = res - emb
            rq = rq + jnp.sum(res * res, axis=1, keepdims=True)
        emb_sum = res0 - res
        h2 = jnp.maximum(
            jnp.dot(emb_sum.astype(jnp.bfloat16), dw1_ref[...],
                    preferred_element_type=jnp.float32)
            + db1_ref[...], 0.0)
        x_hat = (jnp.dot(h2.astype(jnp.bfloat16), dw2_ref[...],
                         preferred_element_type=jnp.float32)
                 + db2_ref[...])
        d = x_hat - x
        recon = jnp.sum(d * d, axis=1, keepdims=True)
        part = part + jnp.sum(recon + (1.0 + _COMMIT) * rq)

    @pl.when(i == 0)
    def _init():
        out_ref[...] = jnp.zeros_like(out_ref)

    out_ref[...] += part.reshape(1, 1)


def kernel(x, enc_W1, enc_b1, enc_W2, enc_b2,
           dec_W1, dec_b1, dec_W2, dec_b2, codebooks, gumbel, gumbel_t):
    del gumbel_t
    cbt = jnp.transpose(codebooks, (0, 2, 1)).astype(jnp.bfloat16)
    cbt2 = jnp.transpose(2.0 * codebooks, (0, 2, 1)).astype(jnp.bfloat16)
    enc_W1 = enc_W1.astype(jnp.bfloat16)
    enc_W2 = enc_W2.astype(jnp.bfloat16)
    dec_W1 = dec_W1.astype(jnp.bfloat16)
    dec_W2 = dec_W2.astype(jnp.bfloat16)
    total = pl.pallas_call(
        _fused_body,
        grid=(_NB,),
        in_specs=[
            pl.BlockSpec((_BM, _D_IN), lambda i: (i, 0)),
            pl.BlockSpec((_D_IN, _D_H), lambda i: (0, 0)),
            pl.BlockSpec((1, _D_H), lambda i: (0, 0)),
            pl.BlockSpec((_D_H, _D_E), lambda i: (0, 0)),
            pl.BlockSpec((1, _D_E), lambda i: (0, 0)),
            pl.BlockSpec((_D_E, _D_H), lambda i: (0, 0)),
            pl.BlockSpec((1, _D_H), lambda i: (0, 0)),
            pl.BlockSpec((_D_H, _D_IN), lambda i: (0, 0)),
            pl.BlockSpec((1, _D_IN), lambda i: (0, 0)),
            pl.BlockSpec((_L, _D_E, _K), lambda i: (0, 0, 0)),
            pl.BlockSpec((_L, _D_E, _K), lambda i: (0, 0, 0)),
            pl.BlockSpec((_L, _BM, _K), lambda i: (0, i, 0)),
        ],
        out_specs=pl.BlockSpec((1, 1), lambda i: (0, 0)),
        out_shape=jax.ShapeDtypeStruct((1, 1), jnp.float32),
        scratch_shapes=[pltpu.VMEM((_L, 1, _K), jnp.float32)],
        compiler_params=pltpu.CompilerParams(
            dimension_semantics=("arbitrary",),
        ),
    )(x, enc_W1, enc_b1.reshape(1, _D_H), enc_W2, enc_b2.reshape(1, _D_E),
      dec_W1, dec_b1.reshape(1, _D_H), dec_W2, dec_b2.reshape(1, _D_IN),
      cbt, cbt2, gumbel)
    return total[0, 0] / _B


# speedup vs baseline: 1.0807x; 1.0807x over previous
import jax
import jax.numpy as jnp
from jax import lax
from jax.experimental import pallas as pl
from jax.experimental.pallas import tpu as pltpu

_B = 16384
_D_IN = 768
_D_H = 2048
_D_E = 256
_K = 1024
_L = 3
_COMMIT = 0.25

_BM = 1024
_NB = _B // _BM


_NCH = 2
_CH = _BM // _NCH


def _fused_body(x_ref, w1_ref, b1_ref, w2_ref, b2_ref,
                dw1_ref, db1_ref, dw2_ref, db2_ref,
                cbt_ref, cbt2_ref, g_ref, out_ref, c2_ref):
    i = pl.program_id(0)

    @pl.when(i == 0)
    def _precompute_c2():
        for l in range(_L):
            cbt32 = cbt_ref[l].astype(jnp.float32)
            c2_ref[l] = jnp.sum(cbt32 * cbt32, axis=0, keepdims=True)
    x = x_ref[...]
    xb = x.astype(jnp.bfloat16)
    h = jnp.maximum(
        jnp.dot(xb, w1_ref[...], preferred_element_type=jnp.float32)
        + b1_ref[...], 0.0)
    res0 = (jnp.dot(h.astype(jnp.bfloat16), w2_ref[...],
                    preferred_element_type=jnp.float32)
            + b2_ref[...])
    res = res0
    rq = jnp.zeros((_BM, 1), jnp.float32)
    for l in range(_L):
        s = (jnp.dot(res.astype(jnp.bfloat16), cbt2_ref[l],
                     preferred_element_type=jnp.float32)
             - c2_ref[l] + g_ref[l]).astype(jnp.bfloat16)
        m = jnp.max(s, axis=1, keepdims=True)
        oh = (s >= m).astype(jnp.bfloat16)
        emb = lax.dot_general(oh, cbt_ref[l], (((1,), (1,)), ((), ())),
                              preferred_element_type=jnp.float32)
        res = res - emb
        rq = rq + jnp.sum(res * res, axis=1, keepdims=True)
    emb_sum = res0 - res
    h2 = jnp.maximum(
        jnp.dot(emb_sum.astype(jnp.bfloat16), dw1_ref[...],
                preferred_element_type=jnp.float32)
        + db1_ref[...], 0.0)
    x_hat = (jnp.dot(h2.astype(jnp.bfloat16), dw2_ref[...],
                     preferred_element_type=jnp.float32)
             + db2_ref[...])
    d = x_hat - x
    recon = jnp.sum(d * d, axis=1, keepdims=True)
    part = jnp.sum(recon + (1.0 + _COMMIT) * rq)

    @pl.when(i == 0)
    def _init():
        out_ref[...] = jnp.zeros_like(out_ref)

    out_ref[...] += part.reshape(1, 1)


def kernel(x, enc_W1, enc_b1, enc_W2, enc_b2,
           dec_W1, dec_b1, dec_W2, dec_b2, codebooks, gumbel, gumbel_t):
    del gumbel_t
    cbt = jnp.transpose(codebooks, (0, 2, 1)).astype(jnp.bfloat16)
    cbt2 = jnp.transpose(2.0 * codebooks, (0, 2, 1)).astype(jnp.bfloat16)
    enc_W1 = enc_W1.astype(jnp.bfloat16)
    enc_W2 = enc_W2.astype(jnp.bfloat16)
    dec_W1 = dec_W1.astype(jnp.bfloat16)
    dec_W2 = dec_W2.astype(jnp.bfloat16)
    total = pl.pallas_call(
        _fused_body,
        grid=(_NB,),
        in_specs=[
            pl.BlockSpec((_BM, _D_IN), lambda i: (i, 0)),
            pl.BlockSpec((_D_IN, _D_H), lambda i: (0, 0)),
            pl.BlockSpec((1, _D_H), lambda i: (0, 0)),
            pl.BlockSpec((_D_H, _D_E), lambda i: (0, 0)),
            pl.BlockSpec((1, _D_E), lambda i: (0, 0)),
            pl.BlockSpec((_D_E, _D_H), lambda i: (0, 0)),
            pl.BlockSpec((1, _D_H), lambda i: (0, 0)),
            pl.BlockSpec((_D_H, _D_IN), lambda i: (0, 0)),
            pl.BlockSpec((1, _D_IN), lambda i: (0, 0)),
            pl.BlockSpec((_L, _D_E, _K), lambda i: (0, 0, 0)),
            pl.BlockSpec((_L, _D_E, _K), lambda i: (0, 0, 0)),
            pl.BlockSpec((_L, _BM, _K), lambda i: (0, i, 0)),
        ],
        out_specs=pl.BlockSpec((1, 1), lambda i: (0, 0)),
        out_shape=jax.ShapeDtypeStruct((1, 1), jnp.float32),
        scratch_shapes=[pltpu.VMEM((_L, 1, _K), jnp.float32)],
        compiler_params=pltpu.CompilerParams(
            dimension_semantics=("arbitrary",),
        ),
    )(x, enc_W1, enc_b1.reshape(1, _D_H), enc_W2, enc_b2.reshape(1, _D_E),
      dec_W1, dec_b1.reshape(1, _D_H), dec_W2, dec_b2.reshape(1, _D_IN),
      cbt, cbt2, gumbel)
    return total[0, 0] / _B


# speedup vs baseline: 1.0846x; 1.0036x over previous
import jax
import jax.numpy as jnp
from jax import lax
from jax.experimental import pallas as pl
from jax.experimental.pallas import tpu as pltpu

_B = 16384
_D_IN = 768
_D_H = 2048
_D_E = 256
_K = 1024
_L = 3
_COMMIT = 0.25

_BM = 1024
_NB = _B // _BM


_NCH = 2
_CH = _BM // _NCH


def _fused_body(x_ref, w1_ref, b1_ref, w2_ref, b2_ref,
                dw1_ref, db1_ref, dw2_ref, db2_ref,
                cbt_ref, cbt2_ref, g_ref, out_ref, c2_ref):
    i = pl.program_id(0)

    @pl.when(i == 0)
    def _precompute_c2():
        for l in range(_L):
            cbt32 = cbt_ref[l].astype(jnp.float32)
            c2_ref[l] = jnp.sum(cbt32 * cbt32, axis=0, keepdims=True)
    x = x_ref[...]
    xb = x.astype(jnp.bfloat16)
    h = jnp.maximum(
        jnp.dot(xb, w1_ref[...], preferred_element_type=jnp.float32)
        + b1_ref[...], 0.0)
    res0 = (jnp.dot(h.astype(jnp.bfloat16), w2_ref[...],
                    preferred_element_type=jnp.float32)
            + b2_ref[...])
    res = res0
    rq = jnp.zeros((_BM, 1), jnp.float32)
    for l in range(_L):
        s = (jnp.dot(res.astype(jnp.bfloat16), cbt2_ref[l],
                     preferred_element_type=jnp.float32)
             - c2_ref[l] + g_ref[l]).astype(jnp.bfloat16)
        m = jnp.max(s, axis=1, keepdims=True)
        oh = (s >= m).astype(jnp.bfloat16)
        emb = lax.dot_general(oh, cbt_ref[l], (((1,), (1,)), ((), ())),
                              preferred_element_type=jnp.float32)
        res = res - emb
        rq = rq + jnp.sum(res * res, axis=1, keepdims=True)
    emb_sum = res0 - res
    h2 = jnp.maximum(
        jnp.dot(emb_sum.astype(jnp.bfloat16), dw1_ref[...],
                preferred_element_type=jnp.float32)
        + db1_ref[...], 0.0)
    x_hat = (jnp.dot(h2.astype(jnp.bfloat16), dw2_ref[...],
                     preferred_element_type=jnp.float32)
             + db2_ref[...])
    d = x_hat - x
    recon = jnp.sum(d * d, axis=1, keepdims=True)
    part = jnp.sum(recon + (1.0 + _COMMIT) * rq)

    @pl.when(i == 0)
    def _init():
        out_ref[...] = jnp.zeros_like(out_ref)

    out_ref[...] += part.reshape(1, 1)


def kernel(x, enc_W1, enc_b1, enc_W2, enc_b2,
           dec_W1, dec_b1, dec_W2, dec_b2, codebooks, gumbel, gumbel_t):
    del gumbel_t
    cbt = jnp.transpose(codebooks, (0, 2, 1)).astype(jnp.bfloat16)
    cbt2 = jnp.transpose(2.0 * codebooks, (0, 2, 1)).astype(jnp.bfloat16)
    enc_W1 = enc_W1.astype(jnp.bfloat16)
    enc_W2 = enc_W2.astype(jnp.bfloat16)
    dec_W1 = dec_W1.astype(jnp.bfloat16)
    dec_W2 = dec_W2.astype(jnp.bfloat16)
    total = pl.pallas_call(
        _fused_body,
        grid=(_NB,),
        in_specs=[
            pl.BlockSpec((_BM, _D_IN), lambda i: (i, 0)),
            pl.BlockSpec((_D_IN, _D_H), lambda i: (0, 0)),
            pl.BlockSpec((1, _D_H), lambda i: (0, 0)),
            pl.BlockSpec((_D_H, _D_E), lambda i: (0, 0)),
            pl.BlockSpec((1, _D_E), lambda i: (0, 0)),
            pl.BlockSpec((_D_E, _D_H), lambda i: (0, 0)),
            pl.BlockSpec((1, _D_H), lambda i: (0, 0)),
            pl.BlockSpec((_D_H, _D_IN), lambda i: (0, 0)),
            pl.BlockSpec((1, _D_IN), lambda i: (0, 0)),
            pl.BlockSpec((_L, _D_E, _K), lambda i: (0, 0, 0)),
            pl.BlockSpec((_L, _D_E, _K), lambda i: (0, 0, 0)),
            pl.BlockSpec((_L, _BM, _K), lambda i: (0, i, 0)),
        ],
        out_specs=pl.BlockSpec((1, 1), lambda i: (0, 0)),
        out_shape=jax.ShapeDtypeStruct((1, 1), jnp.float32),
        scratch_shapes=[pltpu.VMEM((_L, 1, _K), jnp.float32)],
        compiler_params=pltpu.CompilerParams(
            dimension_semantics=("arbitrary",),
            fuse_transposed_lhs_in_matmul=True,
        ),
    )(x, enc_W1, enc_b1.reshape(1, _D_H), enc_W2, enc_b2.reshape(1, _D_E),
      dec_W1, dec_b1.reshape(1, _D_H), dec_W2, dec_b2.reshape(1, _D_IN),
      cbt, cbt2, gumbel)
    return total[0, 0] / _B
